# baseline (speedup 1.0000x reference)
import jax
import jax.numpy as jnp
from jax import lax
from jax.experimental import pallas as pl
from jax.experimental.pallas import tpu as pltpu
from jax.experimental.pallas import tpu_sc as plsc

_N = 10000
_E = 320000
_F = 128
_G = 64
_NC = 2
_NS = 16
_NW = _NC * _NS
_CHUNK = 128
_CPW = 80
_EPAD = _NW * _CHUNK * _CPW
_RPS_PAD = 632
_NPAD = _NS * _RPS_PAD

def _sc_mesh():
    return plsc.VectorSubcoreMesh(core_axis_name="c", subcore_axis_name="s",
                                  num_cores=_NC, num_subcores=_NS)


def _agg_body(vals_hbm, src_hbm, dst_hbm, zeros_hbm, out_hbm,
              idx_s, idx_d, rows, acc, sem):
    c = lax.axis_index("c")
    s = lax.axis_index("s")
    pltpu.sync_copy(zeros_hbm.at[pl.ds(s * _RPS_PAD, _RPS_PAD), :],
                    acc.at[pl.ds(s * _RPS_PAD, _RPS_PAD), :])
    plsc.subcore_barrier()

    base = (c * _NS + s) * _CPW * _CHUNK

    @pl.loop(0, _CPW)
    def _(g):
        off = base + g * _CHUNK
        pltpu.sync_copy(src_hbm.at[pl.ds(off, _CHUNK)], idx_s)
        pltpu.sync_copy(dst_hbm.at[pl.ds(off, _CHUNK)], idx_d)
        pltpu.async_copy(vals_hbm.at[idx_s], rows, sem).wait()
        pltpu.sync_copy(rows, acc.at[idx_d], add=True)

    plsc.subcore_barrier()
    pltpu.sync_copy(acc.at[pl.ds(s * _RPS_PAD, _RPS_PAD), :],
                    out_hbm.at[c, pl.ds(s * _RPS_PAD, _RPS_PAD), :])


def _sc_aggregate(vals, src, dst, zeros):
    k = pl.kernel(
        _agg_body,
        out_type=jax.ShapeDtypeStruct((_NC, _NPAD, _F), jnp.float32),
        mesh=_sc_mesh(),
        scratch_types=[
            pltpu.VMEM((_CHUNK,), jnp.int32),
            pltpu.VMEM((_CHUNK,), jnp.int32),
            pltpu.VMEM((_CHUNK, _F), jnp.float32),
            pltpu.VMEM_SHARED((_NPAD, _F), jnp.float32),
            pltpu.SemaphoreType.DMA,
        ],
    )
    return k(vals, src, dst, zeros)


def _mlp1_body(x_ref, p_ref, w1a, b1a, g1, be1, w1b, b1b, o_ref):
    hin = x_ref[...] + p_ref[0, :_N, :] + p_ref[1, :_N, :]
    a = jnp.dot(hin, w1a[...], preferred_element_type=jnp.float32) + b1a[...]
    m = jnp.mean(a, axis=0, keepdims=True)
    v = jnp.mean(jnp.square(a - m), axis=0, keepdims=True)
    a = (a - m) * lax.rsqrt(v + 1e-5) * g1[...] + be1[...]
    a = jnp.maximum(a, 0.0)
    h = jnp.dot(a, w1b[...], preferred_element_type=jnp.float32) + b1b[...]
    o_ref[...] = jnp.maximum(h, 0.0)


def _mlp2_body(h_ref, q_ref, batch_ref, w2a, b2a, g2, be2, w2b, b2b,
               wc1, bc1, wc2, bc2, h2_ref, gr_ref, lg_ref):
    hin = h_ref[...] + q_ref[0, :_N, :] + q_ref[1, :_N, :]
    a = jnp.dot(hin, w2a[...], preferred_element_type=jnp.float32) + b2a[...]
    m = jnp.mean(a, axis=0, keepdims=True)
    v = jnp.mean(jnp.square(a - m), axis=0, keepdims=True)
    a = (a - m) * lax.rsqrt(v + 1e-5) * g2[...] + be2[...]
    a = jnp.maximum(a, 0.0)
    h2 = jnp.dot(a, w2b[...], preferred_element_type=jnp.float32) + b2b[...]
    h2_ref[...] = h2
    onehot = (batch_ref[...] ==
              lax.broadcasted_iota(jnp.int32, (1, _G), 1)).astype(jnp.float32)
    sums = lax.dot_general(onehot, h2, (((0,), (0,)), ((), ())),
                           preferred_element_type=jnp.float32)
    counts = lax.dot_general(onehot, jnp.ones((_N, 1), jnp.float32),
                             (((0,), (0,)), ((), ())),
                             preferred_element_type=jnp.float32)
    gr = sums / jnp.maximum(counts, 1.0)
    gr_ref[...] = gr
    z = jnp.maximum(
        jnp.dot(gr, wc1[...], preferred_element_type=jnp.float32) + bc1[...],
        0.0)
    lg_ref[...] = jnp.dot(z, wc2[...], preferred_element_type=jnp.float32) + bc2[...]


def _mlp1(x, p, W1a, b1a, bn1g, bn1b, W1b, b1b):
    return pl.pallas_call(
        _mlp1_body,
        out_shape=jax.ShapeDtypeStruct((_N, _F), jnp.float32),
    )(x, p, W1a, b1a.reshape(1, -1), bn1g.reshape(1, -1),
      bn1b.reshape(1, -1), W1b, b1b.reshape(1, -1))


def _mlp2(h, q, batch2d, W2a, b2a, bn2g, bn2b, W2b, b2b, Wc1, bc1, Wc2, bc2):
    return pl.pallas_call(
        _mlp2_body,
        out_shape=(
            jax.ShapeDtypeStruct((_N, _F), jnp.float32),
            jax.ShapeDtypeStruct((_G, _F), jnp.float32),
            jax.ShapeDtypeStruct((_G, 16), jnp.float32),
        ),
    )(h, q, batch2d, W2a, b2a.reshape(1, -1), bn2g.reshape(1, -1),
      bn2b.reshape(1, -1), W2b, b2b.reshape(1, -1), Wc1, bc1.reshape(1, -1),
      Wc2, bc2.reshape(1, -1))


def kernel(x, edge_index, edge_attr, batch,
           W1a, b1a, bn1g, bn1b, W1b, b1b,
           W2a, b2a, bn2g, bn2b, W2b, b2b,
           Wc1, bc1, Wc2, bc2):
    npad = _EPAD - _E
    src = jnp.concatenate([edge_index[0], jnp.zeros((npad,), jnp.int32)])
    dst = jnp.concatenate([edge_index[1], jnp.full((npad,), _N, jnp.int32)])
    zeros = jnp.zeros((_NPAD, _F), jnp.float32)

    p = _sc_aggregate(x, src, dst, zeros)
    h = _mlp1(x, p, W1a, b1a, bn1g, bn1b, W1b, b1b)
    q = _sc_aggregate(h, src, dst, zeros)
    h2, graph_reps, logits = _mlp2(h, q, batch.reshape(-1, 1),
                                   W2a, b2a, bn2g, bn2b, W2b, b2b,
                                   Wc1, bc1, Wc2, bc2)
    return (h2, graph_reps, logits)


# speedup vs baseline: 1.4623x; 1.4623x over previous
import jax
import jax.numpy as jnp
from jax import lax
from jax.experimental import pallas as pl
from jax.experimental.pallas import tpu as pltpu
from jax.experimental.pallas import tpu_sc as plsc

_N = 10000
_E = 320000
_F = 128
_G = 64
_NC = 2
_NS = 16
_NW = _NC * _NS
_CHUNK = 128
_CPW = -(-_E // (_NW * _CHUNK))
_EPAD = _NW * _CHUNK * _CPW
_RPS_PAD = 632
_NPAD = _NS * _RPS_PAD

def _sc_mesh():
    return plsc.VectorSubcoreMesh(core_axis_name="c", subcore_axis_name="s",
                                  num_cores=_NC, num_subcores=_NS)


def _agg_body(vals_hbm, src_hbm, dst_hbm, zeros_hbm, out_hbm,
              idx_s, idx_d, rows, acc, sem):
    c = lax.axis_index("c")
    s = lax.axis_index("s")
    pltpu.sync_copy(zeros_hbm.at[pl.ds(s * _RPS_PAD, _RPS_PAD), :],
                    acc.at[pl.ds(s * _RPS_PAD, _RPS_PAD), :])
    plsc.subcore_barrier()

    base = (c * _NS + s) * _CPW * _CHUNK

    @pl.loop(0, _CPW)
    def _(g):
        off = base + g * _CHUNK
        pltpu.sync_copy(src_hbm.at[pl.ds(off, _CHUNK)], idx_s)
        pltpu.sync_copy(dst_hbm.at[pl.ds(off, _CHUNK)], idx_d)
        pltpu.async_copy(vals_hbm.at[idx_s], rows, sem).wait()
        pltpu.sync_copy(rows, acc.at[idx_d], add=True)

    plsc.subcore_barrier()
    pltpu.sync_copy(acc.at[pl.ds(s * _RPS_PAD, _RPS_PAD), :],
                    out_hbm.at[c, pl.ds(s * _RPS_PAD, _RPS_PAD), :])


def _sc_aggregate(vals, src, dst, zeros):
    k = pl.kernel(
        _agg_body,
        out_type=jax.ShapeDtypeStruct((_NC, _NPAD, _F), jnp.float32),
        mesh=_sc_mesh(),
        scratch_types=[
            pltpu.VMEM((_CHUNK,), jnp.int32),
            pltpu.VMEM((_CHUNK,), jnp.int32),
            pltpu.VMEM((_CHUNK, _F), jnp.float32),
            pltpu.VMEM_SHARED((_NPAD, _F), jnp.float32),
            pltpu.SemaphoreType.DMA,
        ],
    )
    return k(vals, src, dst, zeros)


def _mlp1_body(x_ref, p_ref, w1a, b1a, g1, be1, w1b, b1b, o_ref):
    hin = x_ref[...] + p_ref[0, :_N, :] + p_ref[1, :_N, :]
    a = jnp.dot(hin, w1a[...], preferred_element_type=jnp.float32) + b1a[...]
    m = jnp.mean(a, axis=0, keepdims=True)
    v = jnp.mean(jnp.square(a - m), axis=0, keepdims=True)
    a = (a - m) * lax.rsqrt(v + 1e-5) * g1[...] + be1[...]
    a = jnp.maximum(a, 0.0)
    h = jnp.dot(a, w1b[...], preferred_element_type=jnp.float32) + b1b[...]
    o_ref[...] = jnp.maximum(h, 0.0)


def _mlp2_body(h_ref, q_ref, batch_ref, w2a, b2a, g2, be2, w2b, b2b,
               wc1, bc1, wc2, bc2, h2_ref, gr_ref, lg_ref):
    hin = h_ref[...] + q_ref[0, :_N, :] + q_ref[1, :_N, :]
    a = jnp.dot(hin, w2a[...], preferred_element_type=jnp.float32) + b2a[...]
    m = jnp.mean(a, axis=0, keepdims=True)
    v = jnp.mean(jnp.square(a - m), axis=0, keepdims=True)
    a = (a - m) * lax.rsqrt(v + 1e-5) * g2[...] + be2[...]
    a = jnp.maximum(a, 0.0)
    h2 = jnp.dot(a, w2b[...], preferred_element_type=jnp.float32) + b2b[...]
    h2_ref[...] = h2
    onehot = (batch_ref[...] ==
              lax.broadcasted_iota(jnp.int32, (1, _G), 1)).astype(jnp.float32)
    sums = lax.dot_general(onehot, h2, (((0,), (0,)), ((), ())),
                           preferred_element_type=jnp.float32)
    counts = lax.dot_general(onehot, jnp.ones((_N, 1), jnp.float32),
                             (((0,), (0,)), ((), ())),
                             preferred_element_type=jnp.float32)
    gr = sums / jnp.maximum(counts, 1.0)
    gr_ref[...] = gr
    z = jnp.maximum(
        jnp.dot(gr, wc1[...], preferred_element_type=jnp.float32) + bc1[...],
        0.0)
    lg_ref[...] = jnp.dot(z, wc2[...], preferred_element_type=jnp.float32) + bc2[...]


def _mlp1(x, p, W1a, b1a, bn1g, bn1b, W1b, b1b):
    return pl.pallas_call(
        _mlp1_body,
        out_shape=jax.ShapeDtypeStruct((_N, _F), jnp.float32),
    )(x, p, W1a, b1a.reshape(1, -1), bn1g.reshape(1, -1),
      bn1b.reshape(1, -1), W1b, b1b.reshape(1, -1))


def _mlp2(h, q, batch2d, W2a, b2a, bn2g, bn2b, W2b, b2b, Wc1, bc1, Wc2, bc2):
    return pl.pallas_call(
        _mlp2_body,
        out_shape=(
            jax.ShapeDtypeStruct((_N, _F), jnp.float32),
            jax.ShapeDtypeStruct((_G, _F), jnp.float32),
            jax.ShapeDtypeStruct((_G, 16), jnp.float32),
        ),
    )(h, q, batch2d, W2a, b2a.reshape(1, -1), bn2g.reshape(1, -1),
      bn2b.reshape(1, -1), W2b, b2b.reshape(1, -1), Wc1, bc1.reshape(1, -1),
      Wc2, bc2.reshape(1, -1))


def kernel(x, edge_index, edge_attr, batch,
           W1a, b1a, bn1g, bn1b, W1b, b1b,
           W2a, b2a, bn2g, bn2b, W2b, b2b,
           Wc1, bc1, Wc2, bc2):
    npad = _EPAD - _E
    src = jnp.concatenate([edge_index[0], jnp.zeros((npad,), jnp.int32)])
    dst = jnp.concatenate([edge_index[1], jnp.full((npad,), _N, jnp.int32)])
    zeros = jnp.zeros((_NPAD, _F), jnp.float32)

    p = _sc_aggregate(x, src, dst, zeros)
    h = _mlp1(x, p, W1a, b1a, bn1g, bn1b, W1b, b1b)
    q = _sc_aggregate(h, src, dst, zeros)
    h2, graph_reps, logits = _mlp2(h, q, batch.reshape(-1, 1),
                                   W2a, b2a, bn2g, bn2b, W2b, b2b,
                                   Wc1, bc1, Wc2, bc2)
    return (h2, graph_reps, logits)
